# baseline (speedup 1.0000x reference)
import jax
import jax.numpy as jnp
from jax.experimental import pallas as pl
from jax.experimental.pallas import tpu as pltpu
from jax.experimental.pallas import tpu_sc as plsc

WINDOW = 256
REP = 64


def kernel(tgt, emb_weight):
    batch, hist = tgt.shape
    n = batch * hist
    vocab, dim = emb_weight.shape
    table_rep = jnp.tile(emb_weight, (REP, 1))
    spread = (jnp.arange(n, dtype=jnp.int32) % REP) * vocab
    idx = (tgt.reshape(-1).astype(jnp.int32) + spread).reshape(1, n)

    mesh = plsc.VectorSubcoreMesh(core_axis_name="core",
                                  subcore_axis_name="subcore")

    @pl.kernel(out_type=jax.ShapeDtypeStruct((n, dim), emb_weight.dtype),
               mesh=mesh,
               scratch_types=[pltpu.VMEM_SHARED((vocab * REP, dim), jnp.float32),
                              pltpu.SemaphoreType.DMA])
    def gather_kernel(table_hbm, idx_hbm, out_hbm, table_spmem, sem):
        @pl.when(jax.lax.axis_index("subcore") == 0)
        def _():
            pltpu.sync_copy(table_hbm, table_spmem)

        plsc.subcore_barrier()

        nsplit = 4
        sub = WINDOW // nsplit

        def body(idx_vmem, out_vmem):
            copies = [
                pltpu.async_copy(
                    table_spmem.at[idx_vmem.at[0, pl.ds(j * sub, sub)]],
                    out_vmem.at[pl.ds(j * sub, sub)],
                    sem,
                )
                for j in range(nsplit)
            ]
            for c in copies:
                c.wait()

        pltpu.emit_pipeline(
            body,
            grid=(n // WINDOW,),
            in_specs=[pl.BlockSpec((1, WINDOW), index_map=lambda i: (0, i))],
            out_specs=[pl.BlockSpec((WINDOW, dim), index_map=lambda i: (i, 0))],
            core_axis_name=("core", "subcore"),
            dimension_semantics=(pltpu.PARALLEL,),
        )(idx_hbm, out_hbm)

    out = gather_kernel(table_rep, idx)
    return out.reshape(batch, hist, dim)


# speedup vs baseline: 1.0002x; 1.0002x over previous
import jax
import jax.numpy as jnp
from jax.experimental import pallas as pl
from jax.experimental.pallas import tpu as pltpu
from jax.experimental.pallas import tpu_sc as plsc

WINDOW = 256
REP = 64


def kernel(tgt, emb_weight):
    batch, hist = tgt.shape
    n = batch * hist
    vocab, dim = emb_weight.shape
    table_rep = jnp.tile(emb_weight, (REP, 1))
    spread = (jnp.arange(n, dtype=jnp.int32) % REP) * vocab
    idx = (tgt.reshape(-1).astype(jnp.int32) + spread).reshape(1, n)

    mesh = plsc.VectorSubcoreMesh(core_axis_name="core",
                                  subcore_axis_name="subcore")

    @pl.kernel(out_type=jax.ShapeDtypeStruct((n, dim), emb_weight.dtype),
               mesh=mesh,
               scratch_types=[pltpu.VMEM_SHARED((vocab * REP, dim), jnp.float32),
                              pltpu.SemaphoreType.DMA])
    def gather_kernel(table_hbm, idx_hbm, out_hbm, table_spmem, sem):
        @pl.when(jax.lax.axis_index("subcore") == 0)
        def _():
            pltpu.sync_copy(table_hbm, table_spmem)

        plsc.subcore_barrier()

        def body(idx_vmem, out_vmem):
            pltpu.async_copy(table_spmem.at[idx_vmem.at[0]], out_vmem,
                             sem).wait()

        pltpu.emit_pipeline(
            body,
            grid=(n // WINDOW,),
            in_specs=[pl.BlockSpec((1, WINDOW), index_map=lambda i: (0, i))],
            out_specs=[pl.BlockSpec((WINDOW, dim), index_map=lambda i: (i, 0))],
            core_axis_name=("core", "subcore"),
            dimension_semantics=(pltpu.PARALLEL,),
        )(idx_hbm, out_hbm)

    out = gather_kernel(table_rep, idx)
    return out.reshape(batch, hist, dim)
